# baseline (speedup 1.0000x reference)
import jax
import jax.numpy as jnp
from jax import lax
from jax.experimental import pallas as pl
from jax.experimental.pallas import tpu as pltpu
from jax.experimental.pallas import tpu_sc as plsc

_B = 16384
_K = 1000

_B_SC = 0 * 128
_B_TC = _B - _B_SC
_COLS_TC = 256
_KH = 504

_KS0 = 0
_KS1 = 1234
_KS2 = _KS0 ^ _KS1 ^ 0x1BD11BDA
_KS = (_KS0, _KS1, _KS2)
_ROT = ((13, 15, 26, 6), (17, 29, 16, 24))

_LN2 = 0.6931471805599453
_SQRT2 = 1.4142135623730951


def _rotl(x, r):
    return lax.shift_left(x, r) | lax.shift_right_logical(x, 32 - r)


def _threefry_bits(x1):
    x0 = jnp.zeros_like(x1) + _KS[0]
    x1 = x1 + _KS[1]
    for i in range(5):
        for r in _ROT[i % 2]:
            x0 = x0 + x1
            x1 = _rotl(x1, r) ^ x0
        x0 = x0 + _KS[(i + 1) % 3]
        x1 = x1 + (_KS[(i + 2) % 3] + i + 1)
    return x0 ^ x1


def _bits_to_u(bits):
    fb = lax.shift_right_logical(bits, 9) | 0x3F800000
    return jnp.maximum(lax.bitcast_convert_type(fb, jnp.float32) - 1.0, 1e-20)


def _tc_body(logits_ref, out_ref):
    lg = logits_ref[...]
    m = jnp.max(lg)
    p2 = jnp.exp(lg - m)
    p2 = p2 * p2

    rbase = pl.program_id(0) * _COLS_TC
    row = lax.broadcasted_iota(jnp.int32, (_COLS_TC, _K), 0)
    col = lax.broadcasted_iota(jnp.int32, (_COLS_TC, _K), 1)
    u = _bits_to_u(_threefry_bits((rbase + row) * _K + col))

    w = -jnp.log(u)
    t = p2 / (w * w)
    out_ref[...] = (t / jnp.sum(t, axis=1, keepdims=True)).T


def _tc_part(logits):
    return pl.pallas_call(
        _tc_body,
        grid=(_B_TC // _COLS_TC,),
        in_specs=[pl.BlockSpec((1, _K), lambda i: (0, 0))],
        out_specs=pl.BlockSpec((_K, _COLS_TC), lambda i: (0, i)),
        out_shape=jax.ShapeDtypeStruct((_K, _B), jnp.float32),
    )(logits)


def _neg_log(u):
    ub = lax.bitcast_convert_type(u, jnp.int32)
    e = lax.shift_right_logical(ub, 23) - 127
    mb = (ub & 0x7FFFFF) | 0x3F800000
    m = lax.bitcast_convert_type(mb, jnp.float32)
    big = m > _SQRT2
    e = jnp.where(big, e + 1, e).astype(jnp.float32)
    m = jnp.where(big, m * 0.5, m)
    z = (m - 1.0) / (m + 1.0)
    z2 = z * z
    p = 1.0 + z2 * (1.0 / 3.0 + z2 * (0.2 + z2 * (1.0 / 7.0 + z2 * (1.0 / 9.0))))
    return -(e * _LN2 + (2.0 * z) * p)


def _lane_shuffle(v, stride):
    idx = lax.iota(jnp.int32, 16) ^ stride
    dn = lax.GatherDimensionNumbers(
        offset_dims=(), collapsed_slice_dims=(0,), start_index_map=(0,))
    return lax.gather(v, idx[:, None], dimension_numbers=dn, slice_sizes=(1,),
                      mode=lax.GatherScatterMode.PROMISE_IN_BOUNDS)


def _all_lanes_reduce(v, op):
    for stride in (8, 4, 2, 1):
        v = op(v, _lane_shuffle(v, stride))
    return v


def _sc_body(logits_hbm, out_hbm, p2_v, t_v, acc_v, acc2_v, shr_v):
    c = lax.axis_index("c")
    s = lax.axis_index("s")
    tilecol = c * 8 + (s & 7)
    half = lax.shift_right_logical(s, 3)
    k0 = half * _KH

    p2_v[pl.ds(992, 16)] = jnp.full((16,), -1e30, jnp.float32)
    p2_v[pl.ds(1008, 16)] = jnp.full((16,), -1e30, jnp.float32)
    pltpu.sync_copy(logits_hbm.at[0], p2_v.at[pl.ds(0, _K)])

    def _mx(j, acc):
        return jnp.maximum(acc, p2_v[pl.ds(j * 16, 16)])

    m16 = lax.fori_loop(0, 63, _mx, jnp.full((16,), -1e30, jnp.float32))
    m = _all_lanes_reduce(m16, jnp.maximum)

    def _p2(j, carry):
        v = jnp.exp(p2_v[pl.ds(j * 16, 16)] - m)
        p2_v[pl.ds(j * 16, 16)] = v * v
        return carry

    lax.fori_loop(0, 63, _p2, 0)

    lane1000 = lax.iota(jnp.int32, 16) * _K
    base0 = (_B_TC + tilecol * 128) * _K + k0

    zeros8 = tuple(jnp.zeros((16,), jnp.float32) for _ in range(8))

    @plsc.parallel_loop(0, _KH, carry=zeros8, unroll=2)
    def _cat(i, accs):
        k = k0 + i
        p2s = p2_v[pl.ds(k, 16)][0]
        new = []
        for g in range(8):
            x1 = (base0 + g * 16 * _K + i) + lane1000
            u = _bits_to_u(_threefry_bits(x1))
            w = _neg_log(u)
            t = p2s / (w * w)
            t_v[i, pl.ds(g * 16, 16)] = t
            new.append(accs[g] + t)
        return tuple(new)

    for g in range(8):
        acc_v[pl.ds(g * 16, 16)] = _cat[g]
    pltpu.sync_copy(acc_v, shr_v.at[s])
    plsc.subcore_barrier()
    pltpu.sync_copy(shr_v.at[s ^ 8], acc2_v)

    invs = tuple(1.0 / (_cat[g] + acc2_v[pl.ds(g * 16, 16)])
                 for g in range(8))

    @plsc.parallel_loop(0, _KH, unroll=4)
    def _scale(i):
        for g in range(8):
            t_v[i, pl.ds(g * 16, 16)] = t_v[i, pl.ds(g * 16, 16)] * invs[g]

    colmem = out_hbm.at[:, pl.ds(tilecol * 128, 128)]

    @pl.when(half == 0)
    def _dma0():
        pltpu.sync_copy(t_v.at[pl.ds(0, _KH)], colmem.at[pl.ds(0, _KH)])

    @pl.when(half == 1)
    def _dma1():
        pltpu.sync_copy(t_v.at[pl.ds(0, _K - _KH)],
                        colmem.at[pl.ds(_KH, _K - _KH)])


def _sc_part(logits_vec):
    mesh = plsc.VectorSubcoreMesh(core_axis_name="c", subcore_axis_name="s")
    fn = pl.kernel(
        _sc_body,
        out_type=jax.ShapeDtypeStruct((_K, _B_SC), jnp.float32),
        mesh=mesh,
        scratch_types=[
            pltpu.VMEM((1024,), jnp.float32),
            pltpu.VMEM((_KH, 128), jnp.float32),
            pltpu.VMEM((128,), jnp.float32),
            pltpu.VMEM((128,), jnp.float32),
            pltpu.VMEM_SHARED((16, 128), jnp.float32),
        ],
    )
    return fn(logits_vec)


def kernel(z, logits):
    del z
    out_tc = _tc_part(logits)
    return out_tc.T


# speedup vs baseline: 1.0598x; 1.0598x over previous
import jax
import jax.numpy as jnp
from jax import lax
from jax.experimental import pallas as pl
from jax.experimental.pallas import tpu as pltpu
from jax.experimental.pallas import tpu_sc as plsc

_B = 16384
_K = 1000

_B_SC = 16 * 128
_B_TC = _B - _B_SC
_COLS_TC = 512
_KH = 504

_KS0 = 0
_KS1 = 1234
_KS2 = _KS0 ^ _KS1 ^ 0x1BD11BDA
_KS = (_KS0, _KS1, _KS2)
_ROT = ((13, 15, 26, 6), (17, 29, 16, 24))

_LN2 = 0.6931471805599453
_SQRT2 = 1.4142135623730951


def _rotl(x, r):
    return lax.shift_left(x, r) | lax.shift_right_logical(x, 32 - r)


def _threefry_bits(x1):
    x0 = jnp.zeros_like(x1) + _KS[0]
    x1 = x1 + _KS[1]
    for i in range(5):
        for r in _ROT[i % 2]:
            x0 = x0 + x1
            x1 = _rotl(x1, r) ^ x0
        x0 = x0 + _KS[(i + 1) % 3]
        x1 = x1 + (_KS[(i + 2) % 3] + i + 1)
    return x0 ^ x1


def _bits_to_u(bits):
    fb = lax.shift_right_logical(bits, 9) | 0x3F800000
    return jnp.maximum(lax.bitcast_convert_type(fb, jnp.float32) - 1.0, 1e-20)


def _tc_body(logits_ref, out_ref):
    lg = logits_ref[...]
    m = jnp.max(lg)
    p2 = jnp.exp(lg - m)
    p2 = p2 * p2

    rbase = pl.program_id(0) * _COLS_TC
    row = lax.broadcasted_iota(jnp.int32, (_COLS_TC, _K), 0)
    col = lax.broadcasted_iota(jnp.int32, (_COLS_TC, _K), 1)
    u = _bits_to_u(_threefry_bits((rbase + row) * _K + col))

    w = -jnp.log(u)
    t = p2 / (w * w)
    out_ref[...] = (t / jnp.sum(t, axis=1, keepdims=True)).T


def _tc_part(logits):
    return pl.pallas_call(
        _tc_body,
        grid=(_B_TC // _COLS_TC,),
        in_specs=[pl.BlockSpec((1, _K), lambda i: (0, 0))],
        out_specs=pl.BlockSpec((_K, _COLS_TC), lambda i: (0, i)),
        out_shape=jax.ShapeDtypeStruct((_K, _B), jnp.float32),
    )(logits)


def _neg_log(u):
    ub = lax.bitcast_convert_type(u, jnp.int32)
    e = lax.shift_right_logical(ub, 23) - 127
    mb = (ub & 0x7FFFFF) | 0x3F800000
    m = lax.bitcast_convert_type(mb, jnp.float32)
    big = m > _SQRT2
    e = jnp.where(big, e + 1, e).astype(jnp.float32)
    m = jnp.where(big, m * 0.5, m)
    z = (m - 1.0) / (m + 1.0)
    z2 = z * z
    p = 1.0 + z2 * (1.0 / 3.0 + z2 * (0.2 + z2 * (1.0 / 7.0 + z2 * (1.0 / 9.0))))
    return -(e * _LN2 + (2.0 * z) * p)


def _lane_shuffle(v, stride):
    idx = lax.iota(jnp.int32, 16) ^ stride
    dn = lax.GatherDimensionNumbers(
        offset_dims=(), collapsed_slice_dims=(0,), start_index_map=(0,))
    return lax.gather(v, idx[:, None], dimension_numbers=dn, slice_sizes=(1,),
                      mode=lax.GatherScatterMode.PROMISE_IN_BOUNDS)


def _all_lanes_reduce(v, op):
    for stride in (8, 4, 2, 1):
        v = op(v, _lane_shuffle(v, stride))
    return v


def _sc_body(logits_hbm, out_hbm, p2_v, t_v, acc_v, acc2_v, shr_v):
    c = lax.axis_index("c")
    s = lax.axis_index("s")
    tilecol = c * 8 + (s & 7)
    half = lax.shift_right_logical(s, 3)
    k0 = half * _KH

    p2_v[pl.ds(992, 16)] = jnp.full((16,), -1e30, jnp.float32)
    p2_v[pl.ds(1008, 16)] = jnp.full((16,), -1e30, jnp.float32)
    pltpu.sync_copy(logits_hbm.at[0], p2_v.at[pl.ds(0, _K)])

    def _mx(j, acc):
        return jnp.maximum(acc, p2_v[pl.ds(j * 16, 16)])

    m16 = lax.fori_loop(0, 63, _mx, jnp.full((16,), -1e30, jnp.float32))
    m = _all_lanes_reduce(m16, jnp.maximum)

    def _p2(j, carry):
        v = jnp.exp(p2_v[pl.ds(j * 16, 16)] - m)
        p2_v[pl.ds(j * 16, 16)] = v * v
        return carry

    lax.fori_loop(0, 63, _p2, 0)

    lane1000 = lax.iota(jnp.int32, 16) * _K
    base0 = (_B_TC + tilecol * 128) * _K + k0

    zeros8 = tuple(jnp.zeros((16,), jnp.float32) for _ in range(8))

    @plsc.parallel_loop(0, _KH, carry=zeros8, unroll=2)
    def _cat(i, accs):
        k = k0 + i
        p2s = p2_v[pl.ds(k, 16)][0]
        new = []
        for g in range(8):
            x1 = (base0 + g * 16 * _K + i) + lane1000
            u = _bits_to_u(_threefry_bits(x1))
            w = _neg_log(u)
            t = p2s / (w * w)
            t_v[i, pl.ds(g * 16, 16)] = t
            new.append(accs[g] + t)
        return tuple(new)

    for g in range(8):
        acc_v[pl.ds(g * 16, 16)] = _cat[g]
    pltpu.sync_copy(acc_v, shr_v.at[s])
    plsc.subcore_barrier()
    pltpu.sync_copy(shr_v.at[s ^ 8], acc2_v)

    invs = tuple(1.0 / (_cat[g] + acc2_v[pl.ds(g * 16, 16)])
                 for g in range(8))

    @plsc.parallel_loop(0, _KH, unroll=4)
    def _scale(i):
        for g in range(8):
            t_v[i, pl.ds(g * 16, 16)] = t_v[i, pl.ds(g * 16, 16)] * invs[g]

    colmem = out_hbm.at[:, pl.ds(tilecol * 128, 128)]

    @pl.when(half == 0)
    def _dma0():
        pltpu.sync_copy(t_v.at[pl.ds(0, _KH)], colmem.at[pl.ds(0, _KH)])

    @pl.when(half == 1)
    def _dma1():
        pltpu.sync_copy(t_v.at[pl.ds(0, _K - _KH)],
                        colmem.at[pl.ds(_KH, _K - _KH)])


def _sc_part(logits_vec):
    mesh = plsc.VectorSubcoreMesh(core_axis_name="c", subcore_axis_name="s")
    fn = pl.kernel(
        _sc_body,
        out_type=jax.ShapeDtypeStruct((_K, _B_SC), jnp.float32),
        mesh=mesh,
        scratch_types=[
            pltpu.VMEM((1024,), jnp.float32),
            pltpu.VMEM((_KH, 128), jnp.float32),
            pltpu.VMEM((128,), jnp.float32),
            pltpu.VMEM((128,), jnp.float32),
            pltpu.VMEM_SHARED((16, 128), jnp.float32),
        ],
    )
    return fn(logits_vec)


def kernel(z, logits):
    del z
    out_tc = _tc_part(logits)
    out_sc = _sc_part(logits)
    full_t = lax.dynamic_update_slice(out_tc, out_sc, (0, _B_TC))
    return full_t.T


# speedup vs baseline: 1.0642x; 1.0042x over previous
import jax
import jax.numpy as jnp
from jax import lax
from jax.experimental import pallas as pl
from jax.experimental.pallas import tpu as pltpu
from jax.experimental.pallas import tpu_sc as plsc

_B = 16384
_K = 1000

_B_SC = 16 * 128
_B_TC = _B - _B_SC
_COLS_TC = 1024
_KH = 504

_KS0 = 0
_KS1 = 1234
_KS2 = _KS0 ^ _KS1 ^ 0x1BD11BDA
_KS = (_KS0, _KS1, _KS2)
_ROT = ((13, 15, 26, 6), (17, 29, 16, 24))

_LN2 = 0.6931471805599453
_SQRT2 = 1.4142135623730951


def _rotl(x, r):
    return lax.shift_left(x, r) | lax.shift_right_logical(x, 32 - r)


def _threefry_bits(x1):
    x0 = jnp.zeros_like(x1) + _KS[0]
    x1 = x1 + _KS[1]
    for i in range(5):
        for r in _ROT[i % 2]:
            x0 = x0 + x1
            x1 = _rotl(x1, r) ^ x0
        x0 = x0 + _KS[(i + 1) % 3]
        x1 = x1 + (_KS[(i + 2) % 3] + i + 1)
    return x0 ^ x1


def _bits_to_u(bits):
    fb = lax.shift_right_logical(bits, 9) | 0x3F800000
    return jnp.maximum(lax.bitcast_convert_type(fb, jnp.float32) - 1.0, 1e-20)


def _tc_body(logits_ref, out_ref):
    lg = logits_ref[...]
    m = jnp.max(lg)
    p2 = jnp.exp(lg - m)
    p2 = p2 * p2

    rbase = pl.program_id(0) * _COLS_TC
    row = lax.broadcasted_iota(jnp.int32, (_COLS_TC, _K), 0)
    col = lax.broadcasted_iota(jnp.int32, (_COLS_TC, _K), 1)
    u = _bits_to_u(_threefry_bits((rbase + row) * _K + col))

    w = -jnp.log(u)
    t = p2 / (w * w)
    out_ref[...] = (t / jnp.sum(t, axis=1, keepdims=True)).T


def _tc_part(logits):
    return pl.pallas_call(
        _tc_body,
        grid=(_B_TC // _COLS_TC,),
        in_specs=[pl.BlockSpec((1, _K), lambda i: (0, 0))],
        out_specs=pl.BlockSpec((_K, _COLS_TC), lambda i: (0, i)),
        out_shape=jax.ShapeDtypeStruct((_K, _B), jnp.float32),
    )(logits)


def _neg_log(u):
    ub = lax.bitcast_convert_type(u, jnp.int32)
    e = lax.shift_right_logical(ub, 23) - 127
    mb = (ub & 0x7FFFFF) | 0x3F800000
    m = lax.bitcast_convert_type(mb, jnp.float32)
    big = m > _SQRT2
    e = jnp.where(big, e + 1, e).astype(jnp.float32)
    m = jnp.where(big, m * 0.5, m)
    z = (m - 1.0) / (m + 1.0)
    z2 = z * z
    p = 1.0 + z2 * (1.0 / 3.0 + z2 * (0.2 + z2 * (1.0 / 7.0 + z2 * (1.0 / 9.0))))
    return -(e * _LN2 + (2.0 * z) * p)


def _lane_shuffle(v, stride):
    idx = lax.iota(jnp.int32, 16) ^ stride
    dn = lax.GatherDimensionNumbers(
        offset_dims=(), collapsed_slice_dims=(0,), start_index_map=(0,))
    return lax.gather(v, idx[:, None], dimension_numbers=dn, slice_sizes=(1,),
                      mode=lax.GatherScatterMode.PROMISE_IN_BOUNDS)


def _all_lanes_reduce(v, op):
    for stride in (8, 4, 2, 1):
        v = op(v, _lane_shuffle(v, stride))
    return v


def _sc_body(logits_hbm, out_hbm, p2_v, t_v, acc_v, acc2_v, shr_v):
    c = lax.axis_index("c")
    s = lax.axis_index("s")
    tilecol = c * 8 + (s & 7)
    half = lax.shift_right_logical(s, 3)
    k0 = half * _KH

    p2_v[pl.ds(992, 16)] = jnp.full((16,), -1e30, jnp.float32)
    p2_v[pl.ds(1008, 16)] = jnp.full((16,), -1e30, jnp.float32)
    pltpu.sync_copy(logits_hbm.at[0], p2_v.at[pl.ds(0, _K)])

    def _mx(j, acc):
        return jnp.maximum(acc, p2_v[pl.ds(j * 16, 16)])

    m16 = lax.fori_loop(0, 63, _mx, jnp.full((16,), -1e30, jnp.float32))
    m = _all_lanes_reduce(m16, jnp.maximum)

    def _p2(j, carry):
        v = jnp.exp(p2_v[pl.ds(j * 16, 16)] - m)
        p2_v[pl.ds(j * 16, 16)] = v * v
        return carry

    lax.fori_loop(0, 63, _p2, 0)

    lane1000 = lax.iota(jnp.int32, 16) * _K
    base0 = (_B_TC + tilecol * 128) * _K + k0

    zeros8 = tuple(jnp.zeros((16,), jnp.float32) for _ in range(8))

    @plsc.parallel_loop(0, _KH, carry=zeros8, unroll=2)
    def _cat(i, accs):
        k = k0 + i
        p2s = p2_v[pl.ds(k, 16)][0]
        new = []
        for g in range(8):
            x1 = (base0 + g * 16 * _K + i) + lane1000
            u = _bits_to_u(_threefry_bits(x1))
            w = _neg_log(u)
            t = p2s / (w * w)
            t_v[i, pl.ds(g * 16, 16)] = t
            new.append(accs[g] + t)
        return tuple(new)

    for g in range(8):
        acc_v[pl.ds(g * 16, 16)] = _cat[g]
    pltpu.sync_copy(acc_v, shr_v.at[s])
    plsc.subcore_barrier()
    pltpu.sync_copy(shr_v.at[s ^ 8], acc2_v)

    invs = tuple(1.0 / (_cat[g] + acc2_v[pl.ds(g * 16, 16)])
                 for g in range(8))

    @plsc.parallel_loop(0, _KH, unroll=4)
    def _scale(i):
        for g in range(8):
            t_v[i, pl.ds(g * 16, 16)] = t_v[i, pl.ds(g * 16, 16)] * invs[g]

    colmem = out_hbm.at[:, pl.ds(tilecol * 128, 128)]

    @pl.when(half == 0)
    def _dma0():
        pltpu.sync_copy(t_v.at[pl.ds(0, _KH)], colmem.at[pl.ds(0, _KH)])

    @pl.when(half == 1)
    def _dma1():
        pltpu.sync_copy(t_v.at[pl.ds(0, _K - _KH)],
                        colmem.at[pl.ds(_KH, _K - _KH)])


def _sc_part(logits_vec):
    mesh = plsc.VectorSubcoreMesh(core_axis_name="c", subcore_axis_name="s")
    fn = pl.kernel(
        _sc_body,
        out_type=jax.ShapeDtypeStruct((_K, _B_SC), jnp.float32),
        mesh=mesh,
        scratch_types=[
            pltpu.VMEM((1024,), jnp.float32),
            pltpu.VMEM((_KH, 128), jnp.float32),
            pltpu.VMEM((128,), jnp.float32),
            pltpu.VMEM((128,), jnp.float32),
            pltpu.VMEM_SHARED((16, 128), jnp.float32),
        ],
    )
    return fn(logits_vec)


def kernel(z, logits):
    del z
    out_tc = _tc_part(logits)
    out_sc = _sc_part(logits)
    full_t = lax.dynamic_update_slice(out_tc, out_sc, (0, _B_TC))
    return full_t.T
